# Initial kernel scaffold; baseline (speedup 1.0000x reference)
import jax
import jax.numpy as jnp
from jax.experimental import pallas as pl


def kernel(x, edge_index, batch, W_l, b_l, W_r, W1, b1, W2, b2):
    raise NotImplementedError("write your pallas kernel here")


# speedup vs baseline: 8.6634x; 8.6634x over previous
import functools

import jax
import jax.numpy as jnp
from jax import lax
from jax.experimental import pallas as pl
from jax.experimental.pallas import tpu as pltpu
from jax.experimental.pallas import tpu_sc as plsc

N = 10000
E = 320000
D = 128
H = 128
OUT = 10
G = 64

NC = 2
NS = 16
NTILES = NC * NS
CH = 128
NCHUNK = E // CH
KMAX = -(-NCHUNK // NTILES)
NPAD = 10240
ROWS_PER_TILE = NPAD // NS
DROWS = NPAD // CH

_F32 = jnp.float32


def _sc_scatter(x, e2, zeros_hbm):
    mesh = plsc.VectorSubcoreMesh(
        core_axis_name="c", subcore_axis_name="s", num_cores=NC, num_subcores=NS
    )

    @functools.partial(
        pl.kernel,
        out_type=(
            jax.ShapeDtypeStruct((NC * NPAD, D), _F32),
            jax.ShapeDtypeStruct((NC * DROWS, CH), _F32),
        ),
        mesh=mesh,
        compiler_params=pltpu.CompilerParams(needs_layout_passes=False),
        scratch_types=[
            pltpu.VMEM_SHARED((NPAD, D), _F32),
            pltpu.VMEM((2, CH), jnp.int32),
            pltpu.VMEM((CH, D), _F32),
            pltpu.VMEM((CH, D), _F32),
            pltpu.VMEM((DROWS, CH), _F32),
            pltpu.VMEM((DROWS,), jnp.int32),
            pltpu.VMEM_SHARED((DROWS, CH), _F32),
            pltpu.SemaphoreType.DMA,
        ],
    )
    def k(x_hbm, e2_hbm, z_hbm, part_hbm, deg_hbm, acc, ed_v, rows_v, zrow_v,
          deg_v, idx80_v, dacc, sem):
        c = lax.axis_index("c")
        s = lax.axis_index("s")
        wid = s * NC + c
        base_r = s * ROWS_PER_TILE

        pltpu.sync_copy(z_hbm, zrow_v)
        for j in range(ROWS_PER_TILE // CH):
            pltpu.sync_copy(zrow_v, acc.at[pl.ds(base_r + j * CH, CH)])
        pltpu.sync_copy(z_hbm.at[pl.ds(0, DROWS)], deg_v)
        for i in range(DROWS // 16):
            idx80_v[pl.ds(i * 16, 16)] = lax.iota(jnp.int32, 16) + i * 16

        @pl.when(s == 0)
        def _():
            pltpu.sync_copy(deg_v, dacc)

        plsc.subcore_barrier()

        def body(kk, carry):
            row = kk * NTILES + wid

            @pl.when(row < NCHUNK)
            def _():
                pltpu.sync_copy(e2_hbm.at[row], ed_v)
                pltpu.async_copy(x_hbm.at[ed_v.at[0]], rows_v, sem).wait()
                pltpu.sync_copy(rows_v, acc.at[ed_v.at[1]], add=True)
                ones16 = jnp.full((16,), 1.0, _F32)
                for i in range(CH // 16):
                    d16 = ed_v[1, pl.ds(i * 16, 16)]
                    plsc.addupdate_scatter(
                        deg_v,
                        [lax.shift_right_logical(d16, 7),
                         jnp.bitwise_and(d16, 127)],
                        ones16,
                    )

            return carry

        lax.fori_loop(0, KMAX, body, 0)

        pltpu.sync_copy(deg_v, dacc.at[idx80_v], add=True)
        plsc.subcore_barrier()
        for j in range(ROWS_PER_TILE // CH):
            r0 = base_r + j * CH
            pltpu.sync_copy(acc.at[pl.ds(r0, CH)], zrow_v)
            pltpu.sync_copy(zrow_v, part_hbm.at[pl.ds(c * NPAD + r0, CH)])

        @pl.when(s == 0)
        def _():
            pltpu.sync_copy(dacc, deg_v)
            pltpu.sync_copy(deg_v, deg_hbm.at[pl.ds(c * DROWS, DROWS)])

    return k(x, e2, zeros_hbm)


def _tc_dense(part, degp, x, batch_r, W_l, b_l, W_r, W1, b1, W2, b2):
    hi = jax.lax.Precision.HIGHEST

    def body(part_ref, deg_ref, x_ref, b_ref, wl_ref, bl_ref, wr_ref, w1_ref,
             b1_ref, w2_ref, b2_ref, emb_ref, logp_ref):
        agg = part_ref[0] + part_ref[1]
        dh = deg_ref[0] + deg_ref[1]
        sel = (lax.broadcasted_iota(jnp.int32, (NPAD, CH), 0) % CH
               == lax.broadcasted_iota(jnp.int32, (NPAD, CH), 1))
        dhrep = jnp.broadcast_to(dh[:, None, :], (DROWS, CH, CH)).reshape(NPAD, CH)
        deg = jnp.sum(jnp.where(sel, dhrep, 0.0), axis=1, keepdims=True)[:N]
        mean = agg / jnp.maximum(deg, 1.0)
        emb = (jnp.dot(mean, wl_ref[...], precision=hi)
               + bl_ref[...]
               + jnp.dot(x_ref[...], wr_ref[...], precision=hi))
        emb_ref[...] = emb
        h = jnp.maximum(emb, 0.0)
        gids = lax.broadcasted_iota(jnp.int32, (G, 1), 0)
        onehot_t = (gids == b_ref[...]).astype(_F32)
        pooled_sum = jnp.dot(onehot_t, h, precision=hi)
        cnt = jnp.sum(onehot_t, axis=1, keepdims=True)
        pooled = pooled_sum / jnp.maximum(cnt, 1.0)
        z1 = jnp.maximum(jnp.dot(pooled, w1_ref[...], precision=hi) + b1_ref[...], 0.0)
        z = jnp.dot(z1, w2_ref[...], precision=hi) + b2_ref[...]
        m = jnp.max(z, axis=1, keepdims=True)
        lse = jnp.log(jnp.sum(jnp.exp(z - m), axis=1, keepdims=True)) + m
        logp_ref[...] = z - lse

    return pl.pallas_call(
        body,
        out_shape=(
            jax.ShapeDtypeStruct((N, H), _F32),
            jax.ShapeDtypeStruct((G, OUT), _F32),
        ),
    )(part, degp, x, batch_r, W_l, b_l, W_r, W1, b1, W2, b2)


def kernel(x, edge_index, batch, W_l, b_l, W_r, W1, b1, W2, b2):
    src = edge_index[0].reshape(NCHUNK, CH)
    dst = edge_index[1].reshape(NCHUNK, CH)
    e2 = jnp.stack([src, dst], axis=1)
    zeros_hbm = jnp.zeros((CH, D), _F32)
    part, degp = _sc_scatter(x, e2, zeros_hbm)
    part = part.reshape(NC, NPAD, D)[:, :N]
    degp = degp.reshape(NC, DROWS, CH)
    emb, logp = _tc_dense(
        part, degp, x, batch.reshape(1, N), W_l, b_l.reshape(1, H), W_r,
        W1, b1.reshape(1, 50), W2, b2.reshape(1, OUT))
    return emb, logp


# speedup vs baseline: 14.2119x; 1.6404x over previous
import functools

import jax
import jax.numpy as jnp
from jax import lax
from jax.experimental import pallas as pl
from jax.experimental.pallas import tpu as pltpu
from jax.experimental.pallas import tpu_sc as plsc

N = 10000
E = 320000
D = 128
H = 128
OUT = 10
G = 64

NC = 2
NS = 16
NTILES = NC * NS
CH = 128
MACRO = 4 * CH
NMACRO = E // MACRO
KMAX = -(-NMACRO // NTILES)
NPAD = 10240
ROWS_PER_TILE = NPAD // NS
DROWS = NPAD // CH

_F32 = jnp.float32


def _sc_scatter(x, e2, zeros_hbm):
    mesh = plsc.VectorSubcoreMesh(
        core_axis_name="c", subcore_axis_name="s", num_cores=NC, num_subcores=NS
    )

    @functools.partial(
        pl.kernel,
        out_type=(
            jax.ShapeDtypeStruct((NC * NPAD, D), _F32),
            jax.ShapeDtypeStruct((NC * DROWS, CH), _F32),
        ),
        mesh=mesh,
        compiler_params=pltpu.CompilerParams(
            needs_layout_passes=False, use_tc_tiling_on_sc=False),
        scratch_types=[
            pltpu.VMEM_SHARED((NPAD, D), _F32),
            pltpu.VMEM((8, CH), jnp.int32),
            pltpu.VMEM((8, CH), jnp.int32),
            pltpu.VMEM((CH, D), _F32),
            pltpu.VMEM((CH, D), _F32),
            pltpu.VMEM((DROWS, CH), _F32),
            pltpu.VMEM((DROWS,), jnp.int32),
            pltpu.VMEM_SHARED((DROWS, CH), _F32),
            pltpu.SemaphoreType.DMA,
            pltpu.SemaphoreType.DMA,
        ],
    )
    def k(x_hbm, e2_hbm, z_hbm, part_hbm, deg_hbm, acc, ed0_v, ed1_v, rows0_v,
          rows1_v, deg_v, idx80_v, dacc, sem0, sem1):
        zrow_v = rows0_v
        c = lax.axis_index("c")
        s = lax.axis_index("s")
        wid = s * NC + c
        base_r = s * ROWS_PER_TILE

        pltpu.sync_copy(z_hbm, zrow_v)
        for j in range(ROWS_PER_TILE // CH):
            pltpu.sync_copy(zrow_v, acc.at[pl.ds(base_r + j * CH, CH)])
        pltpu.sync_copy(z_hbm.at[pl.ds(0, DROWS)], deg_v)
        for i in range(DROWS // 16):
            idx80_v[pl.ds(i * 16, 16)] = lax.iota(jnp.int32, 16) + i * 16

        @pl.when(s == 0)
        def _():
            pltpu.sync_copy(deg_v, dacc)

        plsc.subcore_barrier()

        rbufs = ((rows0_v, sem0), (rows1_v, sem1))
        ones16 = jnp.full((16,), 1.0, _F32)

        def valid(kk):
            return kk * NTILES + wid < NMACRO

        def idx_copy(kk, ed_v):
            @pl.when(valid(kk))
            def _():
                pltpu.sync_copy(e2_hbm.at[kk * NTILES + wid], ed_v)

        def fire(kk, ed_v, j, b):
            rows_v, sem = rbufs[b]

            @pl.when(valid(kk))
            def _():
                pltpu.async_copy(x_hbm.at[ed_v.at[j]], rows_v, sem)

        def drain(kk, ed_v, j, b):
            rows_v, sem = rbufs[b]

            @pl.when(valid(kk))
            def _():
                pltpu.make_async_copy(x_hbm.at[ed_v.at[j]], rows_v, sem).wait()
                pltpu.sync_copy(rows_v, acc.at[ed_v.at[4 + j]], add=True)

        def deghist(kk, ed_v):
            @pl.when(valid(kk))
            def _():
                for j in range(4):
                    for i in range(CH // 16):
                        d16 = ed_v[4 + j, pl.ds(i * 16, 16)]
                        plsc.addupdate_scatter(
                            deg_v,
                            [lax.shift_right_logical(d16, 7),
                             jnp.bitwise_and(d16, 127)],
                            ones16,
                        )

        idx_copy(0, ed0_v)
        fire(0, ed0_v, 0, 0)

        def body(k2, carry):
            m0 = 2 * k2
            m1 = m0 + 1
            fire(m0, ed0_v, 1, 1)
            deghist(m0, ed0_v)
            drain(m0, ed0_v, 0, 0)
            fire(m0, ed0_v, 2, 0)
            drain(m0, ed0_v, 1, 1)
            fire(m0, ed0_v, 3, 1)
            idx_copy(m1, ed1_v)
            drain(m0, ed0_v, 2, 0)
            fire(m1, ed1_v, 0, 0)
            deghist(m1, ed1_v)
            drain(m0, ed0_v, 3, 1)
            fire(m1, ed1_v, 1, 1)
            drain(m1, ed1_v, 0, 0)
            fire(m1, ed1_v, 2, 0)
            idx_copy(m0 + 2, ed0_v)
            drain(m1, ed1_v, 1, 1)
            fire(m1, ed1_v, 3, 1)
            drain(m1, ed1_v, 2, 0)
            fire(m0 + 2, ed0_v, 0, 0)
            drain(m1, ed1_v, 3, 1)
            return carry

        lax.fori_loop(0, KMAX // 2, body, 0)

        pltpu.sync_copy(deg_v, dacc.at[idx80_v], add=True)
        plsc.subcore_barrier()
        for j in range(ROWS_PER_TILE // CH):
            r0 = base_r + j * CH
            pltpu.sync_copy(acc.at[pl.ds(r0, CH)], zrow_v)
            pltpu.sync_copy(zrow_v, part_hbm.at[pl.ds(c * NPAD + r0, CH)])

        @pl.when(s == 0)
        def _():
            pltpu.sync_copy(dacc, deg_v)
            pltpu.sync_copy(deg_v, deg_hbm.at[pl.ds(c * DROWS, DROWS)])

    return k(x, e2, zeros_hbm)


def _tc_dense(part, degp, x, batch_r, W_l, b_l, W_r, W1, b1, W2, b2):
    hi = jax.lax.Precision.HIGHEST

    def body(part_ref, deg_ref, x_ref, b_ref, wl_ref, bl_ref, wr_ref, w1_ref,
             b1_ref, w2_ref, b2_ref, emb_ref, logp_ref):
        agg = part_ref[0] + part_ref[1]
        dh = deg_ref[0] + deg_ref[1]
        sel = (lax.broadcasted_iota(jnp.int32, (NPAD, CH), 0) % CH
               == lax.broadcasted_iota(jnp.int32, (NPAD, CH), 1))
        dhrep = jnp.broadcast_to(dh[:, None, :], (DROWS, CH, CH)).reshape(NPAD, CH)
        deg = jnp.sum(jnp.where(sel, dhrep, 0.0), axis=1, keepdims=True)[:N]
        mean = agg / jnp.maximum(deg, 1.0)
        emb = (jnp.dot(mean, wl_ref[...], precision=hi)
               + bl_ref[...]
               + jnp.dot(x_ref[...], wr_ref[...], precision=hi))
        emb_ref[...] = emb
        h = jnp.maximum(emb, 0.0)
        gids = lax.broadcasted_iota(jnp.int32, (G, 1), 0)
        onehot_t = (gids == b_ref[...]).astype(_F32)
        pooled_sum = jnp.dot(onehot_t, h, precision=hi)
        cnt = jnp.sum(onehot_t, axis=1, keepdims=True)
        pooled = pooled_sum / jnp.maximum(cnt, 1.0)
        z1 = jnp.maximum(jnp.dot(pooled, w1_ref[...], precision=hi) + b1_ref[...], 0.0)
        z = jnp.dot(z1, w2_ref[...], precision=hi) + b2_ref[...]
        m = jnp.max(z, axis=1, keepdims=True)
        lse = jnp.log(jnp.sum(jnp.exp(z - m), axis=1, keepdims=True)) + m
        logp_ref[...] = z - lse

    return pl.pallas_call(
        body,
        out_shape=(
            jax.ShapeDtypeStruct((N, H), _F32),
            jax.ShapeDtypeStruct((G, OUT), _F32),
        ),
    )(part, degp, x, batch_r, W_l, b_l, W_r, W1, b1, W2, b2)


def kernel(x, edge_index, batch, W_l, b_l, W_r, W1, b1, W2, b2):
    src = edge_index[0].reshape(NMACRO, 4, CH)
    dst = edge_index[1].reshape(NMACRO, 4, CH)
    e2 = jnp.concatenate([src, dst], axis=1)
    zeros_hbm = jnp.zeros((CH, D), _F32)
    part, degp = _sc_scatter(x, e2, zeros_hbm)
    part = part.reshape(NC, NPAD, D)[:, :N]
    degp = degp.reshape(NC, DROWS, CH)
    emb, logp = _tc_dense(
        part, degp, x, batch.reshape(1, N), W_l, b_l.reshape(1, H), W_r,
        W1, b1.reshape(1, 50), W2, b2.reshape(1, OUT))
    return emb, logp


# speedup vs baseline: 14.4880x; 1.0194x over previous
import functools

import jax
import jax.numpy as jnp
from jax import lax
from jax.experimental import pallas as pl
from jax.experimental.pallas import tpu as pltpu
from jax.experimental.pallas import tpu_sc as plsc

N = 10000
E = 320000
D = 128
H = 128
OUT = 10
G = 64

NC = 2
NS = 16
NTILES = NC * NS
CH = 128
MACRO = 4 * CH
NMACRO = E // MACRO
KMAX = -(-NMACRO // NTILES)
NPAD = 10240
ROWS_PER_TILE = NPAD // NS
DROWS = NPAD // CH

_F32 = jnp.float32


def _sc_scatter(x, e2, zeros_hbm):
    mesh = plsc.VectorSubcoreMesh(
        core_axis_name="c", subcore_axis_name="s", num_cores=NC, num_subcores=NS
    )

    @functools.partial(
        pl.kernel,
        out_type=(
            jax.ShapeDtypeStruct((NC * NPAD, D), _F32),
            jax.ShapeDtypeStruct((NC * DROWS, CH), _F32),
        ),
        mesh=mesh,
        compiler_params=pltpu.CompilerParams(
            needs_layout_passes=False, use_tc_tiling_on_sc=False),
        scratch_types=[
            pltpu.VMEM_SHARED((NPAD, D), _F32),
            pltpu.VMEM((8, CH), jnp.int32),
            pltpu.VMEM((8, CH), jnp.int32),
            pltpu.VMEM((CH, D), _F32),
            pltpu.VMEM((CH, D), _F32),
            pltpu.VMEM((DROWS, CH), _F32),
            pltpu.VMEM((DROWS,), jnp.int32),
            pltpu.VMEM_SHARED((DROWS, CH), _F32),
            pltpu.SemaphoreType.DMA,
            pltpu.SemaphoreType.DMA,
        ],
    )
    def k(x_hbm, e2_hbm, z_hbm, part_hbm, deg_hbm, acc, ed0_v, ed1_v, rows0_v,
          rows1_v, deg_v, idx80_v, dacc, sem0, sem1):
        zrow_v = rows0_v
        c = lax.axis_index("c")
        s = lax.axis_index("s")
        wid = s * NC + c
        base_r = s * ROWS_PER_TILE

        pltpu.sync_copy(z_hbm, zrow_v)
        for j in range(ROWS_PER_TILE // CH):
            pltpu.sync_copy(zrow_v, acc.at[pl.ds(base_r + j * CH, CH)])
        pltpu.sync_copy(z_hbm.at[pl.ds(0, DROWS)], deg_v)
        for i in range(DROWS // 16):
            idx80_v[pl.ds(i * 16, 16)] = lax.iota(jnp.int32, 16) + i * 16

        @pl.when(s == 0)
        def _():
            pltpu.sync_copy(deg_v, dacc)

        plsc.subcore_barrier()

        rbufs = ((rows0_v, sem0), (rows1_v, sem1))
        ones16 = jnp.full((16,), 1.0, _F32)

        def valid(kk):
            return kk * NTILES + wid < NMACRO

        def idx_copy(kk, ed_v):
            @pl.when(valid(kk))
            def _():
                pltpu.sync_copy(e2_hbm.at[kk * NTILES + wid], ed_v)

        def fire(kk, ed_v, j, b):
            rows_v, sem = rbufs[b]

            @pl.when(valid(kk))
            def _():
                pltpu.async_copy(x_hbm.at[ed_v.at[j]], rows_v, sem)

        def drain(kk, ed_v, j, b):
            rows_v, sem = rbufs[b]

            @pl.when(valid(kk))
            def _():
                pltpu.make_async_copy(x_hbm.at[ed_v.at[j]], rows_v, sem).wait()
                pltpu.sync_copy(rows_v, acc.at[ed_v.at[4 + j]], add=True)

        def deghist(kk, ed_v):
            @pl.when(valid(kk))
            def _():
                for j in range(4):
                    for i in range(CH // 16):
                        d16 = ed_v[4 + j, pl.ds(i * 16, 16)]
                        plsc.addupdate_scatter(
                            deg_v,
                            [lax.shift_right_logical(d16, 7),
                             jnp.bitwise_and(d16, 127)],
                            ones16,
                        )

        idx_copy(0, ed0_v)
        fire(0, ed0_v, 0, 0)

        def body(k2, carry):
            m0 = 2 * k2
            m1 = m0 + 1
            fire(m0, ed0_v, 1, 1)
            deghist(m0, ed0_v)
            drain(m0, ed0_v, 0, 0)
            fire(m0, ed0_v, 2, 0)
            drain(m0, ed0_v, 1, 1)
            fire(m0, ed0_v, 3, 1)
            idx_copy(m1, ed1_v)
            drain(m0, ed0_v, 2, 0)
            fire(m1, ed1_v, 0, 0)
            deghist(m1, ed1_v)
            drain(m0, ed0_v, 3, 1)
            fire(m1, ed1_v, 1, 1)
            drain(m1, ed1_v, 0, 0)
            fire(m1, ed1_v, 2, 0)
            idx_copy(m0 + 2, ed0_v)
            drain(m1, ed1_v, 1, 1)
            fire(m1, ed1_v, 3, 1)
            drain(m1, ed1_v, 2, 0)
            fire(m0 + 2, ed0_v, 0, 0)
            drain(m1, ed1_v, 3, 1)
            return carry

        lax.fori_loop(0, KMAX // 2, body, 0)

        pltpu.sync_copy(deg_v, dacc.at[idx80_v], add=True)
        plsc.subcore_barrier()
        for j in range(ROWS_PER_TILE // CH):
            r0 = base_r + j * CH
            pltpu.sync_copy(acc.at[pl.ds(r0, CH)], zrow_v)
            pltpu.sync_copy(zrow_v, part_hbm.at[pl.ds(c * NPAD + r0, CH)])

        @pl.when(s == 0)
        def _():
            pltpu.sync_copy(dacc, deg_v)
            pltpu.sync_copy(deg_v, deg_hbm.at[pl.ds(c * DROWS, DROWS)])

    return k(x, e2, zeros_hbm)


def _tc_self(x, batch_r, W_r, b_l):
    hi = jax.lax.Precision.HIGHEST

    def body(x_ref, b_ref, wr_ref, bl_ref, xr_ref, cnt_ref):
        xr_ref[...] = jnp.dot(x_ref[...], wr_ref[...], precision=hi) + bl_ref[...]
        gids = lax.broadcasted_iota(jnp.int32, (G, 1), 0)
        onehot_t = (gids == b_ref[...]).astype(_F32)
        cnt_ref[...] = jnp.sum(onehot_t, axis=1, keepdims=True)

    return pl.pallas_call(
        body,
        out_shape=(
            jax.ShapeDtypeStruct((N, H), _F32),
            jax.ShapeDtypeStruct((G, 1), _F32),
        ),
    )(x, batch_r, W_r, b_l)


def _tc_dense(part, degp, xr, cnt, batch_r, W_l, W1, b1, W2, b2):
    hi = jax.lax.Precision.HIGHEST

    def body(part_ref, deg_ref, xr_ref, cnt_ref, b_ref, wl_ref, w1_ref,
             b1_ref, w2_ref, b2_ref, emb_ref, logp_ref):
        agg = part_ref[0] + part_ref[1]
        dh = deg_ref[0] + deg_ref[1]
        sel = (lax.broadcasted_iota(jnp.int32, (NPAD, CH), 0) % CH
               == lax.broadcasted_iota(jnp.int32, (NPAD, CH), 1))
        dhrep = jnp.broadcast_to(dh[:, None, :], (DROWS, CH, CH)).reshape(NPAD, CH)
        deg = jnp.sum(jnp.where(sel, dhrep, 0.0), axis=1, keepdims=True)[:N]
        mean = agg / jnp.maximum(deg, 1.0)
        emb = jnp.dot(mean, wl_ref[...], precision=hi) + xr_ref[...]
        emb_ref[...] = emb
        h = jnp.maximum(emb, 0.0)
        gids = lax.broadcasted_iota(jnp.int32, (G, 1), 0)
        onehot_t = (gids == b_ref[...]).astype(_F32)
        pooled_sum = jnp.dot(onehot_t, h, precision=hi)
        pooled = pooled_sum / jnp.maximum(cnt_ref[...], 1.0)
        z1 = jnp.maximum(jnp.dot(pooled, w1_ref[...], precision=hi) + b1_ref[...], 0.0)
        z = jnp.dot(z1, w2_ref[...], precision=hi) + b2_ref[...]
        m = jnp.max(z, axis=1, keepdims=True)
        lse = jnp.log(jnp.sum(jnp.exp(z - m), axis=1, keepdims=True)) + m
        logp_ref[...] = z - lse

    return pl.pallas_call(
        body,
        out_shape=(
            jax.ShapeDtypeStruct((N, H), _F32),
            jax.ShapeDtypeStruct((G, OUT), _F32),
        ),
    )(part, degp, xr, cnt, batch_r, W_l, W1, b1, W2, b2)


def kernel(x, edge_index, batch, W_l, b_l, W_r, W1, b1, W2, b2):
    src = edge_index[0].reshape(NMACRO, 4, CH)
    dst = edge_index[1].reshape(NMACRO, 4, CH)
    e2 = jnp.concatenate([src, dst], axis=1)
    zeros_hbm = jnp.zeros((CH, D), _F32)
    batch_r = batch.reshape(1, N)
    part, degp = _sc_scatter(x, e2, zeros_hbm)
    xr, cnt = _tc_self(x, batch_r, W_r, b_l.reshape(1, H))
    part = part.reshape(NC, NPAD, D)[:, :N]
    degp = degp.reshape(NC, DROWS, CH)
    emb, logp = _tc_dense(
        part, degp, xr, cnt, batch_r, W_l,
        W1, b1.reshape(1, 50), W2, b2.reshape(1, OUT))
    return emb, logp


# speedup vs baseline: 14.5928x; 1.0072x over previous
import functools

import jax
import jax.numpy as jnp
from jax import lax
from jax.experimental import pallas as pl
from jax.experimental.pallas import tpu as pltpu
from jax.experimental.pallas import tpu_sc as plsc

N = 10000
E = 320000
D = 128
H = 128
OUT = 10
G = 64

NC = 2
NS = 16
NTILES = NC * NS
CH = 128
MACRO = 4 * CH
NMACRO = E // MACRO
KMAX = -(-NMACRO // NTILES)
NPAD = 10240
ROWS_PER_TILE = NPAD // NS
DROWS = NPAD // CH

_F32 = jnp.float32


def _sc_scatter(x, e2, zeros_hbm):
    mesh = plsc.VectorSubcoreMesh(
        core_axis_name="c", subcore_axis_name="s", num_cores=NC, num_subcores=NS
    )

    @functools.partial(
        pl.kernel,
        out_type=(
            jax.ShapeDtypeStruct((NC * NPAD, D), _F32),
            jax.ShapeDtypeStruct((NC * DROWS, CH), _F32),
        ),
        mesh=mesh,
        compiler_params=pltpu.CompilerParams(
            needs_layout_passes=False, use_tc_tiling_on_sc=False),
        scratch_types=[
            pltpu.VMEM_SHARED((NPAD, D), _F32),
            pltpu.VMEM((8, CH), jnp.int32),
            pltpu.VMEM((8, CH), jnp.int32),
            pltpu.VMEM((CH, D), _F32),
            pltpu.VMEM((CH, D), _F32),
            pltpu.VMEM((DROWS, CH), _F32),
            pltpu.VMEM((DROWS,), jnp.int32),
            pltpu.VMEM_SHARED((DROWS, CH), _F32),
            pltpu.SemaphoreType.DMA,
            pltpu.SemaphoreType.DMA,
        ],
    )
    def k(x_hbm, e2_hbm, z_hbm, part_hbm, deg_hbm, acc, ed0_v, ed1_v, rows0_v,
          rows1_v, deg_v, idx80_v, dacc, sem0, sem1):
        zrow_v = rows0_v
        c = lax.axis_index("c")
        s = lax.axis_index("s")
        wid = s * NC + c
        base_r = s * ROWS_PER_TILE

        pltpu.sync_copy(z_hbm, zrow_v)
        for j in range(ROWS_PER_TILE // CH):
            pltpu.sync_copy(zrow_v, acc.at[pl.ds(base_r + j * CH, CH)])
        pltpu.sync_copy(z_hbm.at[pl.ds(0, DROWS)], deg_v)
        for i in range(DROWS // 16):
            idx80_v[pl.ds(i * 16, 16)] = lax.iota(jnp.int32, 16) + i * 16

        @pl.when(s == 0)
        def _():
            pltpu.sync_copy(deg_v, dacc)

        plsc.subcore_barrier()

        rbufs = ((rows0_v, sem0), (rows1_v, sem1))
        ones16 = jnp.full((16,), 1.0, _F32)

        def valid(kk):
            return kk * NTILES + wid < NMACRO

        def idx_copy(kk, ed_v):
            @pl.when(valid(kk))
            def _():
                pltpu.sync_copy(e2_hbm.at[kk * NTILES + wid], ed_v)

        def fire(kk, ed_v, j, b):
            rows_v, sem = rbufs[b]

            @pl.when(valid(kk))
            def _():
                pltpu.async_copy(x_hbm.at[ed_v.at[j]], rows_v, sem)

        def drain(kk, ed_v, j, b):
            rows_v, sem = rbufs[b]

            @pl.when(valid(kk))
            def _():
                pltpu.make_async_copy(x_hbm.at[ed_v.at[j]], rows_v, sem).wait()
                pltpu.sync_copy(rows_v, acc.at[ed_v.at[4 + j]], add=True)

        def deghist(kk, ed_v):
            @pl.when(valid(kk))
            def _():
                for j in range(4):
                    for i in range(CH // 16):
                        d16 = ed_v[4 + j, pl.ds(i * 16, 16)]
                        plsc.addupdate_scatter(
                            deg_v,
                            [lax.shift_right_logical(d16, 7),
                             jnp.bitwise_and(d16, 127)],
                            ones16,
                        )

        idx_copy(0, ed0_v)
        fire(0, ed0_v, 0, 0)

        def body(k2, carry):
            m0 = 2 * k2
            m1 = m0 + 1
            fire(m0, ed0_v, 1, 1)
            deghist(m0, ed0_v)
            drain(m0, ed0_v, 0, 0)
            fire(m0, ed0_v, 2, 0)
            drain(m0, ed0_v, 1, 1)
            fire(m0, ed0_v, 3, 1)
            idx_copy(m1, ed1_v)
            drain(m0, ed0_v, 2, 0)
            fire(m1, ed1_v, 0, 0)
            deghist(m1, ed1_v)
            drain(m0, ed0_v, 3, 1)
            fire(m1, ed1_v, 1, 1)
            drain(m1, ed1_v, 0, 0)
            fire(m1, ed1_v, 2, 0)
            idx_copy(m0 + 2, ed0_v)
            drain(m1, ed1_v, 1, 1)
            fire(m1, ed1_v, 3, 1)
            drain(m1, ed1_v, 2, 0)
            fire(m0 + 2, ed0_v, 0, 0)
            drain(m1, ed1_v, 3, 1)
            return carry

        lax.fori_loop(0, KMAX // 2, body, 0)

        pltpu.sync_copy(deg_v, dacc.at[idx80_v], add=True)
        plsc.subcore_barrier()
        for j in range(ROWS_PER_TILE // CH):
            r0 = base_r + j * CH
            pltpu.sync_copy(acc.at[pl.ds(r0, CH)], zrow_v)
            pltpu.sync_copy(zrow_v, part_hbm.at[pl.ds(c * NPAD + r0, CH)])

        @pl.when(s == 0)
        def _():
            pltpu.sync_copy(dacc, deg_v)
            pltpu.sync_copy(deg_v, deg_hbm.at[pl.ds(c * DROWS, DROWS)])

    return k(x, e2, zeros_hbm)


NB = 8
BN = NPAD // NB


def _tc_self(x_pad, batch_p, W_r, b_l):
    hi = jax.lax.Precision.HIGHEST

    def body(x_ref, b_ref, wr_ref, bl_ref, xr_ref, cnt_ref):
        xr_ref[...] = jnp.dot(x_ref[...], wr_ref[...], precision=hi) + bl_ref[...]
        gids = lax.broadcasted_iota(jnp.int32, (G, 1), 0)
        onehot_t = (gids == b_ref[...]).astype(_F32)
        cnt_ref[...] = jnp.sum(onehot_t, axis=1, keepdims=True)

    return pl.pallas_call(
        body,
        out_shape=(
            jax.ShapeDtypeStruct((NPAD, H), _F32),
            jax.ShapeDtypeStruct((G, 1), _F32),
        ),
    )(x_pad, batch_p, W_r, b_l)


def _tc_dense(part, degp, xr, cnt, batch_p, W_l, W1, b1, W2, b2):
    hi = jax.lax.Precision.HIGHEST

    def body(part_ref, deg_ref, xr_ref, cnt_ref, b_ref, wl_ref, w1_ref,
             b1_ref, w2_ref, b2_ref, emb_ref, logp_ref, pool_acc):
        i = pl.program_id(0)
        agg = part_ref[0] + part_ref[1]
        hrows = pl.ds(i * (BN // CH), BN // CH)
        dhb = deg_ref[0, hrows] + deg_ref[1, hrows]
        sel = (lax.broadcasted_iota(jnp.int32, (BN, CH), 0) % CH
               == lax.broadcasted_iota(jnp.int32, (BN, CH), 1))
        dhrep = jnp.broadcast_to(
            dhb[:, None, :], (BN // CH, CH, CH)).reshape(BN, CH)
        deg = jnp.sum(jnp.where(sel, dhrep, 0.0), axis=1, keepdims=True)
        mean = agg / jnp.maximum(deg, 1.0)
        emb = jnp.dot(mean, wl_ref[...], precision=hi) + xr_ref[...]
        emb_ref[...] = emb
        h = jnp.maximum(emb, 0.0)
        gids = lax.broadcasted_iota(jnp.int32, (G, 1), 0)
        onehot_t = (gids == b_ref[...]).astype(_F32)
        psum = jnp.dot(onehot_t, h, precision=hi)

        @pl.when(i == 0)
        def _():
            pool_acc[...] = jnp.zeros_like(pool_acc)

        pool_acc[...] += psum

        @pl.when(i == NB - 1)
        def _():
            pooled = pool_acc[...] / jnp.maximum(cnt_ref[...], 1.0)
            z1 = jnp.maximum(
                jnp.dot(pooled, w1_ref[...], precision=hi) + b1_ref[...], 0.0)
            z = jnp.dot(z1, w2_ref[...], precision=hi) + b2_ref[...]
            m = jnp.max(z, axis=1, keepdims=True)
            lse = jnp.log(jnp.sum(jnp.exp(z - m), axis=1, keepdims=True)) + m
            logp_ref[...] = z - lse

    full = lambda *shape: pl.BlockSpec(shape, lambda i: (0,) * len(shape))
    return pl.pallas_call(
        body,
        grid=(NB,),
        in_specs=[
            pl.BlockSpec((NC, BN, D), lambda i: (0, i, 0)),
            full(NC, DROWS, CH),
            pl.BlockSpec((BN, H), lambda i: (i, 0)),
            full(G, 1),
            pl.BlockSpec((1, BN), lambda i: (0, i)),
            full(D, H),
            full(H, 50),
            full(1, 50),
            full(50, OUT),
            full(1, OUT),
        ],
        out_specs=(
            pl.BlockSpec((BN, H), lambda i: (i, 0)),
            pl.BlockSpec((G, OUT), lambda i: (0, 0)),
        ),
        out_shape=(
            jax.ShapeDtypeStruct((NPAD, H), _F32),
            jax.ShapeDtypeStruct((G, OUT), _F32),
        ),
        scratch_shapes=[pltpu.VMEM((G, H), _F32)],
    )(part, degp, xr, cnt, batch_p, W_l, W1, b1, W2, b2)


def kernel(x, edge_index, batch, W_l, b_l, W_r, W1, b1, W2, b2):
    src = edge_index[0].reshape(NMACRO, 4, CH)
    dst = edge_index[1].reshape(NMACRO, 4, CH)
    e2 = jnp.concatenate([src, dst], axis=1)
    zeros_hbm = jnp.zeros((CH, D), _F32)
    x_pad = jnp.concatenate([x, jnp.zeros((NPAD - N, D), _F32)], axis=0)
    batch_p = jnp.concatenate(
        [batch, jnp.full((NPAD - N,), G, jnp.int32)]).reshape(1, NPAD)
    part, degp = _sc_scatter(x, e2, zeros_hbm)
    xr, cnt = _tc_self(x_pad, batch_p, W_r, b_l.reshape(1, H))
    part = part.reshape(NC, NPAD, D)
    degp = degp.reshape(NC, DROWS, CH)
    emb, logp = _tc_dense(
        part, degp, xr, cnt, batch_p, W_l,
        W1, b1.reshape(1, 50), W2, b2.reshape(1, OUT))
    return emb[:N], logp


# speedup vs baseline: 15.4150x; 1.0563x over previous
import functools

import jax
import jax.numpy as jnp
from jax import lax
from jax.experimental import pallas as pl
from jax.experimental.pallas import tpu as pltpu
from jax.experimental.pallas import tpu_sc as plsc

N = 10000
E = 320000
D = 128
H = 128
OUT = 10
G = 64

NC = 2
NS = 16
NTILES = NC * NS
CH = 128
MACRO = 4 * CH
NMACRO = E // MACRO
KMAX = -(-NMACRO // NTILES)
NPAD = 10240
ROWS_PER_TILE = NPAD // NS
DROWS = NPAD // CH

_F32 = jnp.float32


def _sc_scatter(x, e2, zeros_hbm):
    mesh = plsc.VectorSubcoreMesh(
        core_axis_name="c", subcore_axis_name="s", num_cores=NC, num_subcores=NS
    )

    @functools.partial(
        pl.kernel,
        out_type=(
            jax.ShapeDtypeStruct((NC * NPAD, D), _F32),
            jax.ShapeDtypeStruct((NC * DROWS, CH), _F32),
        ),
        mesh=mesh,
        compiler_params=pltpu.CompilerParams(
            needs_layout_passes=False, use_tc_tiling_on_sc=False),
        scratch_types=[
            pltpu.VMEM_SHARED((NPAD, D), _F32),
            pltpu.VMEM((8, CH), jnp.int32),
            pltpu.VMEM((8, CH), jnp.int32),
            pltpu.VMEM((8, CH), jnp.int32),
            pltpu.VMEM((8, CH), jnp.int32),
            pltpu.VMEM((CH, D), _F32),
            pltpu.VMEM((CH, D), _F32),
            pltpu.VMEM((DROWS, CH), _F32),
            pltpu.VMEM((DROWS,), jnp.int32),
            pltpu.VMEM_SHARED((DROWS, CH), _F32),
            pltpu.SemaphoreType.DMA,
            pltpu.SemaphoreType.DMA,
            pltpu.SemaphoreType.DMA,
            pltpu.SemaphoreType.DMA,
            pltpu.SemaphoreType.DMA,
            pltpu.SemaphoreType.DMA,
        ],
    )
    def k(x_hbm, e2_hbm, z_hbm, part_hbm, deg_hbm, acc, ed0_v, ed1_v, ed2_v,
          ed3_v, rows0_v, rows1_v, deg_v, idx80_v, dacc, sem0, sem1,
          si0, si1, si2, si3):
        zrow_v = rows0_v
        c = lax.axis_index("c")
        s = lax.axis_index("s")
        wid = s * NC + c
        base_r = s * ROWS_PER_TILE

        pltpu.sync_copy(z_hbm, zrow_v)
        for j in range(ROWS_PER_TILE // CH):
            pltpu.sync_copy(zrow_v, acc.at[pl.ds(base_r + j * CH, CH)])
        pltpu.sync_copy(z_hbm.at[pl.ds(0, DROWS)], deg_v)
        for i in range(DROWS // 16):
            idx80_v[pl.ds(i * 16, 16)] = lax.iota(jnp.int32, 16) + i * 16

        @pl.when(s == 0)
        def _():
            pltpu.sync_copy(deg_v, dacc)

        plsc.subcore_barrier()

        rbufs = ((rows0_v, sem0), (rows1_v, sem1))
        ebufs = ((ed0_v, si0), (ed1_v, si1), (ed2_v, si2), (ed3_v, si3))
        ones16 = jnp.full((16,), 1.0, _F32)

        def valid(kk):
            return kk * NTILES + wid < NMACRO

        def idx_fire(kk, e):
            ed_v, si = ebufs[e]

            @pl.when(valid(kk))
            def _():
                pltpu.async_copy(e2_hbm.at[kk * NTILES + wid], ed_v, si)

        def idx_wait(kk, e):
            ed_v, si = ebufs[e]

            @pl.when(valid(kk))
            def _():
                pltpu.make_async_copy(
                    e2_hbm.at[kk * NTILES + wid], ed_v, si).wait()

        def fire(kk, e, j, b):
            ed_v, _ = ebufs[e]
            rows_v, sem = rbufs[b]

            @pl.when(valid(kk))
            def _():
                pltpu.async_copy(x_hbm.at[ed_v.at[j]], rows_v, sem)

        def drain(kk, e, j, b):
            ed_v, _ = ebufs[e]
            rows_v, sem = rbufs[b]

            @pl.when(valid(kk))
            def _():
                pltpu.make_async_copy(x_hbm.at[ed_v.at[j]], rows_v, sem).wait()
                pltpu.sync_copy(rows_v, acc.at[ed_v.at[4 + j]], add=True)

        def deghist(kk, e):
            ed_v, _ = ebufs[e]

            @pl.when(valid(kk))
            def _():
                for j in range(4):
                    for i in range(CH // 16):
                        d16 = ed_v[4 + j, pl.ds(i * 16, 16)]
                        plsc.addupdate_scatter(
                            deg_v,
                            [lax.shift_right_logical(d16, 7),
                             jnp.bitwise_and(d16, 127)],
                            ones16,
                        )

        idx_fire(0, 0)
        idx_wait(0, 0)
        idx_fire(1, 1)
        fire(0, 0, 0, 0)

        def body(k4, carry):
            a = 4 * k4
            b, c, d, n0, n1 = a + 1, a + 2, a + 3, a + 4, a + 5
            fire(a, 0, 1, 1)
            deghist(a, 0)
            idx_wait(b, 1)
            idx_fire(c, 2)
            drain(a, 0, 0, 0)
            fire(a, 0, 2, 0)
            drain(a, 0, 1, 1)
            fire(a, 0, 3, 1)
            drain(a, 0, 2, 0)
            fire(b, 1, 0, 0)
            deghist(b, 1)
            idx_fire(d, 3)
            drain(a, 0, 3, 1)
            fire(b, 1, 1, 1)
            drain(b, 1, 0, 0)
            fire(b, 1, 2, 0)
            idx_wait(c, 2)
            drain(b, 1, 1, 1)
            fire(b, 1, 3, 1)
            drain(b, 1, 2, 0)
            fire(c, 2, 0, 0)
            deghist(c, 2)
            idx_fire(n0, 0)
            drain(b, 1, 3, 1)
            fire(c, 2, 1, 1)
            drain(c, 2, 0, 0)
            fire(c, 2, 2, 0)
            idx_wait(d, 3)
            drain(c, 2, 1, 1)
            fire(c, 2, 3, 1)
            drain(c, 2, 2, 0)
            fire(d, 3, 0, 0)
            deghist(d, 3)
            idx_fire(n1, 1)
            drain(c, 2, 3, 1)
            fire(d, 3, 1, 1)
            drain(d, 3, 0, 0)
            fire(d, 3, 2, 0)
            idx_wait(n0, 0)
            drain(d, 3, 1, 1)
            fire(d, 3, 3, 1)
            drain(d, 3, 2, 0)
            fire(n0, 0, 0, 0)
            drain(d, 3, 3, 1)
            return carry

        lax.fori_loop(0, KMAX // 4, body, 0)

        pltpu.sync_copy(deg_v, dacc.at[idx80_v], add=True)
        plsc.subcore_barrier()
        for j in range(ROWS_PER_TILE // CH):
            r0 = base_r + j * CH
            pltpu.sync_copy(acc.at[pl.ds(r0, CH)], zrow_v)
            pltpu.sync_copy(zrow_v, part_hbm.at[pl.ds(c * NPAD + r0, CH)])

        @pl.when(s == 0)
        def _():
            pltpu.sync_copy(dacc, deg_v)
            pltpu.sync_copy(deg_v, deg_hbm.at[pl.ds(c * DROWS, DROWS)])

    return k(x, e2, zeros_hbm)


NB = 8
BN = NPAD // NB


def _tc_self(x, batch_p, W_r, b_l):
    hi = jax.lax.Precision.HIGHEST

    def body(x_ref, b_ref, wr_ref, bl_ref, xr_ref, cnt_ref):
        xr_ref[...] = jnp.dot(x_ref[...], wr_ref[...], precision=hi) + bl_ref[...]
        gids = lax.broadcasted_iota(jnp.int32, (G, 1), 0)
        onehot_t = (gids == b_ref[...]).astype(_F32)
        cnt_ref[...] = jnp.sum(onehot_t, axis=1, keepdims=True)

    return pl.pallas_call(
        body,
        out_shape=(
            jax.ShapeDtypeStruct((N, H), _F32),
            jax.ShapeDtypeStruct((G, 1), _F32),
        ),
    )(x, batch_p, W_r, b_l)


def _tc_dense(part, degp, xr, cnt, batch_p, W_l, W1, b1, W2, b2):
    hi = jax.lax.Precision.HIGHEST

    def body(part_ref, deg_ref, xr_ref, cnt_ref, b_ref, wl_ref, w1_ref,
             b1_ref, w2_ref, b2_ref, emb_ref, logp_ref, pool_acc):
        i = pl.program_id(0)
        agg = part_ref[0] + part_ref[1]
        hrows = pl.ds(i * (BN // CH), BN // CH)
        dhb = deg_ref[0, hrows] + deg_ref[1, hrows]
        sel = (lax.broadcasted_iota(jnp.int32, (BN, CH), 0) % CH
               == lax.broadcasted_iota(jnp.int32, (BN, CH), 1))
        dhrep = jnp.broadcast_to(
            dhb[:, None, :], (BN // CH, CH, CH)).reshape(BN, CH)
        deg = jnp.sum(jnp.where(sel, dhrep, 0.0), axis=1, keepdims=True)
        mean = agg / jnp.maximum(deg, 1.0)
        emb = jnp.dot(mean, wl_ref[...], precision=hi) + xr_ref[...]
        emb_ref[...] = emb
        rid = lax.broadcasted_iota(jnp.int32, (BN, 1), 0) + i * BN
        h = jnp.where(rid < N, jnp.maximum(emb, 0.0), 0.0)
        gids = lax.broadcasted_iota(jnp.int32, (G, 1), 0)
        onehot_t = (gids == b_ref[...]).astype(_F32)
        psum = jnp.dot(onehot_t, h, precision=hi)

        @pl.when(i == 0)
        def _():
            pool_acc[...] = jnp.zeros_like(pool_acc)

        pool_acc[...] += psum

        @pl.when(i == NB - 1)
        def _():
            pooled = pool_acc[...] / jnp.maximum(cnt_ref[...], 1.0)
            z1 = jnp.maximum(
                jnp.dot(pooled, w1_ref[...], precision=hi) + b1_ref[...], 0.0)
            z = jnp.dot(z1, w2_ref[...], precision=hi) + b2_ref[...]
            m = jnp.max(z, axis=1, keepdims=True)
            lse = jnp.log(jnp.sum(jnp.exp(z - m), axis=1, keepdims=True)) + m
            logp_ref[...] = z - lse

    full = lambda *shape: pl.BlockSpec(shape, lambda i: (0,) * len(shape))
    return pl.pallas_call(
        body,
        grid=(NB,),
        in_specs=[
            pl.BlockSpec((NC, BN, D), lambda i: (0, i, 0)),
            full(NC, DROWS, CH),
            pl.BlockSpec((BN, H), lambda i: (i, 0)),
            full(G, 1),
            pl.BlockSpec((1, BN), lambda i: (0, i)),
            full(D, H),
            full(H, 50),
            full(1, 50),
            full(50, OUT),
            full(1, OUT),
        ],
        out_specs=(
            pl.BlockSpec((BN, H), lambda i: (i, 0)),
            pl.BlockSpec((G, OUT), lambda i: (0, 0)),
        ),
        out_shape=(
            jax.ShapeDtypeStruct((N, H), _F32),
            jax.ShapeDtypeStruct((G, OUT), _F32),
        ),
        scratch_shapes=[pltpu.VMEM((G, H), _F32)],
    )(part, degp, xr, cnt, batch_p, W_l, W1, b1, W2, b2)


def kernel(x, edge_index, batch, W_l, b_l, W_r, W1, b1, W2, b2):
    src = edge_index[0].reshape(NMACRO, 4, CH)
    dst = edge_index[1].reshape(NMACRO, 4, CH)
    e2 = jnp.concatenate([src, dst], axis=1)
    zeros_hbm = jnp.zeros((CH, D), _F32)
    batch_p = jnp.concatenate(
        [batch, jnp.full((NPAD - N,), G, jnp.int32)]).reshape(1, NPAD)
    part, degp = _sc_scatter(x, e2, zeros_hbm)
    xr, cnt = _tc_self(x, batch_p, W_r, b_l.reshape(1, H))
    part = part.reshape(NC, NPAD, D)
    degp = degp.reshape(NC, DROWS, CH)
    emb, logp = _tc_dense(
        part, degp, xr, cnt, batch_p, W_l,
        W1, b1.reshape(1, 50), W2, b2.reshape(1, OUT))
    return emb, logp
